# baseline (speedup 1.0000x reference)
import functools

import jax
import jax.numpy as jnp
from jax import lax
from jax.experimental import pallas as pl
from jax.experimental.pallas import tpu as pltpu
from jax.experimental.pallas import tpu_sc as plsc

_LOG2 = 0.6931471805599453

_NC = 2
_NS = 16
_L = 16

_W = 96
_WT = 80
_WPC = 20
_CH = _W * _WPC
_NBUF = 3


def _ssp(v):
    return jnp.maximum(v, 0.0) + jnp.log1p(jnp.exp(-jnp.abs(v))) - _LOG2


def _node_body(x_ref, wi_ref, bi_ref, wj_ref, bj_ref, xi_ref, xj0_ref,
               xj1_ref):
    xa = _ssp(x_ref[...])
    xi = _ssp(jnp.dot(xa, wi_ref[...], preferred_element_type=jnp.float32)
              + bi_ref[...])
    xj = _ssp(jnp.dot(xa, wj_ref[...], preferred_element_type=jnp.float32)
              + bj_ref[...])
    xi_ref[...] = xi
    xj0_ref[...] = xj[:, :32]
    xj1_ref[...] = xj[:, 32:]


def _node_transform(x, wi, bi, wj, bj, bn):
    n, d = x.shape
    grid = (n // bn,)
    return pl.pallas_call(
        _node_body,
        grid=grid,
        in_specs=[
            pl.BlockSpec((bn, d), lambda i: (i, 0)),
            pl.BlockSpec((d, d), lambda i: (0, 0)),
            pl.BlockSpec((d,), lambda i: (0,)),
            pl.BlockSpec((d, d), lambda i: (0, 0)),
            pl.BlockSpec((d,), lambda i: (0,)),
        ],
        out_specs=[
            pl.BlockSpec((bn, d), lambda i: (i, 0)),
            pl.BlockSpec((bn, 32), lambda i: (i, 0)),
            pl.BlockSpec((bn, 32), lambda i: (i, 0)),
        ],
        out_shape=[
            jax.ShapeDtypeStruct((n, d), jnp.float32),
            jax.ShapeDtypeStruct((n, 32), jnp.float32),
            jax.ShapeDtypeStruct((n, 32), jnp.float32),
        ],
    )(x, wi, bi, wj, bj)


def _g_body(f0_ref, f1_ref, f2_ref, f3_ref, wa_ref, g0_ref, g1_ref):
    outs0, outs1 = [], []
    for fr in (f0_ref, f1_ref, f2_ref, f3_ref):
        g = jnp.dot(fr[...], wa_ref[...],
                    preferred_element_type=jnp.float32)
        outs0.append(g[:, :32])
        outs1.append(g[:, 32:])
    g0_ref[...] = jnp.concatenate(outs0, axis=1)
    g1_ref[...] = jnp.concatenate(outs1, axis=1)


def _g_transform(f_ij, w_att, bq):
    e, r = f_ij.shape
    d = w_att.shape[1]
    nblk = e // 4 // bq
    grid = (nblk,)

    def fmap(a):
        return lambda i: (i + a * nblk, 0)

    return pl.pallas_call(
        _g_body,
        grid=grid,
        in_specs=[
            pl.BlockSpec((bq, r), fmap(0)),
            pl.BlockSpec((bq, r), fmap(1)),
            pl.BlockSpec((bq, r), fmap(2)),
            pl.BlockSpec((bq, r), fmap(3)),
            pl.BlockSpec((r, d), lambda i: (0, 0)),
        ],
        out_specs=[
            pl.BlockSpec((bq, 128), lambda i: (i, 0)),
            pl.BlockSpec((bq, 128), lambda i: (i, 0)),
        ],
        out_shape=[
            jax.ShapeDtypeStruct((e // 4, 128), jnp.float32),
            jax.ShapeDtypeStruct((e // 4, 128), jnp.float32),
        ],
    )(f_ij, f_ij, f_ij, f_ij, w_att)


def _make_sc_edge(n, e):
    ept = e // _NS
    nwin = ept // _W
    tail = ept - nwin * _W
    nchunk = nwin // _WPC
    assert nwin == nchunk * _WPC and nwin >= _NBUF
    assert tail in (0, _WT) and tail % 8 == 0
    rpt = -((-n // _NS) // 8) * 8
    rlast = n - rpt * (_NS - 1)
    assert rlast > 0

    mesh = plsc.VectorSubcoreMesh(
        core_axis_name="c", subcore_axis_name="s",
        num_cores=_NC, num_subcores=_NS)

    @functools.partial(
        pl.kernel,
        mesh=mesh,
        compiler_params=pltpu.CompilerParams(use_tc_tiling_on_sc=False),
        out_type=jax.ShapeDtypeStruct((n, 128), jnp.float32),
        scratch_types=[
            pltpu.VMEM((_CH,), jnp.int32),
            pltpu.VMEM((_CH,), jnp.int32),
            pltpu.VMEM((_NBUF, _W), jnp.int32),
            pltpu.VMEM((_WT,), jnp.int32),
            pltpu.VMEM((_NBUF, _W, 32), jnp.float32),
            pltpu.VMEM((_NBUF, _W, 32), jnp.float32),
            pltpu.VMEM_SHARED((n, 32), jnp.float32),
            pltpu.SemaphoreType.DMA((_NBUF,)),
            pltpu.SemaphoreType.DMA((_NBUF,)),
        ],
    )
    def sc_edge(xj0_hbm, xj1_hbm, g0_hbm, g1_hbm,
                pair_hbm, zeros_hbm, out_hbm,
                ci_v, cj_v, idxi_v, ti_v, rows_v, g_v, acc, sem_in, sem_sc):
        c = lax.axis_index("c")
        s = lax.axis_index("s")

        r0 = s * rpt

        @pl.when(s < _NS - 1)
        def _():
            pltpu.sync_copy(zeros_hbm.at[pl.ds(r0, rpt)],
                            acc.at[pl.ds(r0, rpt)])

        @pl.when(s == _NS - 1)
        def _():
            pltpu.sync_copy(zeros_hbm.at[pl.ds(r0, rlast)],
                            acc.at[pl.ds(r0, rlast)])

        plsc.subcore_barrier()

        ebase = s * ept

        def _drain(buf, slot, sem):
            pltpu.make_async_copy(zeros_hbm.at[pl.ds(0, _W)],
                                  buf.at[slot], sem).wait()

        def _run(table_hbm, gq_hbm, cidx):
            qa = s // 4
            gbase = ebase - qa * (e // 4)
            gcol = qa * 32

            def chunk_body(q, carry):
                cb = ebase + q * _CH
                pltpu.sync_copy(pair_hbm.at[0, pl.ds(cb, _CH)], ci_v)
                pltpu.sync_copy(pair_hbm.at[1, pl.ds(cb, _CH)], cj_v)

                def win_body(t, carry2):
                    gw = q * _WPC + t

                    @pl.when(t < _WPC)
                    def _():
                        slot = gw % _NBUF

                        @pl.when(gw >= _NBUF)
                        def _():
                            _drain(rows_v, slot, sem_sc.at[slot])

                        for i in range(_W // _L):
                            sl = pl.ds(i * _L, _L)
                            src = pl.ds(t * _W + i * _L, _L)
                            idxi_v[slot, sl] = ci_v[src]
                        pltpu.async_copy(
                            table_hbm.at[cj_v.at[pl.ds(t * _W, _W)]],
                            rows_v.at[slot], sem_in.at[slot])
                        pltpu.async_copy(
                            gq_hbm.at[pl.ds(gbase + q * _CH + t * _W, _W),
                                      pl.ds(gcol, 32)],
                            g_v.at[slot], sem_in.at[slot])

                    @pl.when(t >= 2)
                    def _():
                        slot = (gw - 2) % _NBUF
                        _drain(rows_v, slot, sem_in.at[slot])
                        _drain(g_v, slot, sem_in.at[slot])

                        def mul_body(jb, carry3):
                            for jj in range(8):
                                j = jb * 8 + jj
                                for h in range(2):
                                    sl = pl.ds(h * _L, _L)
                                    rows_v[slot, j, sl] = (
                                        rows_v[slot, j, sl] * g_v[slot, j, sl])
                            return carry3

                        lax.fori_loop(0, _W // 8, mul_body, 0)
                        pltpu.async_copy(rows_v.at[slot],
                                         acc.at[idxi_v.at[slot]],
                                         sem_sc.at[slot], add=True)

                    return carry2

                lax.fori_loop(0, _WPC + 2, win_body, 0)
                return carry

            lax.fori_loop(0, nchunk, chunk_body, 0)
            for b in range(_NBUF):
                _drain(rows_v, b, sem_sc.at[b])

            if tail:
                tb = ebase + nwin * _W
                pltpu.sync_copy(pair_hbm.at[0, pl.ds(tb, _WT)], ti_v)
                pltpu.sync_copy(pair_hbm.at[1, pl.ds(tb, _WT)],
                                cj_v.at[pl.ds(0, _WT)])
                pltpu.async_copy(table_hbm.at[cj_v.at[pl.ds(0, _WT)]],
                                 rows_v.at[0, pl.ds(0, _WT)],
                                 sem_in.at[0]).wait()
                pltpu.sync_copy(
                    gq_hbm.at[pl.ds(gbase + nwin * _W, _WT),
                              pl.ds(gcol, 32)],
                    g_v.at[0, pl.ds(0, _WT)])
                for j in range(_WT):
                    for h in range(2):
                        sl = pl.ds(h * _L, _L)
                        rows_v[0, j, sl] = rows_v[0, j, sl] * g_v[0, j, sl]
                pltpu.sync_copy(rows_v.at[0, pl.ds(0, _WT)],
                                acc.at[ti_v], add=True)

            plsc.subcore_barrier()

            @pl.when(s < _NS - 1)
            def _():
                pltpu.sync_copy(acc.at[pl.ds(r0, rpt)],
                                out_hbm.at[pl.ds(r0, rpt),
                                           pl.ds(cidx * 32, 32)])

            @pl.when(s == _NS - 1)
            def _():
                pltpu.sync_copy(acc.at[pl.ds(r0, rlast)],
                                out_hbm.at[pl.ds(r0, rlast),
                                           pl.ds(cidx * 32, 32)])

        @pl.when(c == 0)
        def _():
            _run(xj0_hbm, g0_hbm, 0)

        @pl.when(c == 1)
        def _():
            _run(xj1_hbm, g1_hbm, 1)

    return sc_edge


def _post_body(x_ref, xi_ref, xjp_ref, dw_ref, db_ref, rw_ref, rb_ref,
               wv_ref, bv_ref, gate_ref, o_ref):
    m = xi_ref[...] + xjp_ref[...][:, :64]
    for l in range(3):
        t = _ssp(jnp.dot(m, dw_ref[l], preferred_element_type=jnp.float32)
                 + db_ref[l])
        m = m + jnp.dot(t, rw_ref[l], preferred_element_type=jnp.float32) \
              + rb_ref[l]
    o_ref[...] = (gate_ref[...] * x_ref[...]
                  + jnp.dot(_ssp(m), wv_ref[...],
                            preferred_element_type=jnp.float32)
                  + bv_ref[...])


def _post_transform(x, xi, xjp, dw, db, rw, rb, wv, bv, gate, bn):
    n, d = x.shape
    nres = dw.shape[0]
    grid = (n // bn,)
    return pl.pallas_call(
        _post_body,
        grid=grid,
        in_specs=[
            pl.BlockSpec((bn, d), lambda i: (i, 0)),
            pl.BlockSpec((bn, d), lambda i: (i, 0)),
            pl.BlockSpec((bn, 128), lambda i: (i, 0)),
            pl.BlockSpec((nres, d, d), lambda i: (0, 0, 0)),
            pl.BlockSpec((nres, d), lambda i: (0, 0)),
            pl.BlockSpec((nres, d, d), lambda i: (0, 0, 0)),
            pl.BlockSpec((nres, d), lambda i: (0, 0)),
            pl.BlockSpec((d, d), lambda i: (0, 0)),
            pl.BlockSpec((d,), lambda i: (0,)),
            pl.BlockSpec((d,), lambda i: (0,)),
        ],
        out_specs=pl.BlockSpec((bn, d), lambda i: (i, 0)),
        out_shape=jax.ShapeDtypeStruct((n, d), jnp.float32),
    )(x, xi, xjp, dw, db, rw, rb, wv, bv, gate)


def kernel(atomic_embedding, pair_indices, f_ij, W_att, Wi, bi, Wj, bj,
           Wv, bv, dW, db, rW, rb, gate):
    x = atomic_embedding
    n, d = x.shape
    e = f_ij.shape[0]
    idx_i = pair_indices[0]
    idx_j = pair_indices[1]

    bn = min(2000, n)
    bq = min(2000, e // 4)
    xi, xj0, xj1 = _node_transform(x, Wi, bi, Wj, bj, bn=bn)
    g0, g1 = _g_transform(f_ij, W_att, bq=bq)

    zeros = jnp.zeros((n, 32), jnp.float32)

    sc_edge = _make_sc_edge(n, e)
    xjp = sc_edge(xj0, xj1, g0, g1, pair_indices, zeros)

    return _post_transform(x, xi, xjp, dW, db, rW, rb, Wv, bv, gate, bn=bn)


# speedup vs baseline: 1.2417x; 1.2417x over previous
import functools

import jax
import jax.numpy as jnp
from jax import lax
from jax.experimental import pallas as pl
from jax.experimental.pallas import tpu as pltpu
from jax.experimental.pallas import tpu_sc as plsc

_LOG2 = 0.6931471805599453

_NC = 2
_NS = 16
_L = 16

_W = 80
_WPC = 25
_CH = _W * _WPC
_NBUF = 3


def _ssp(v):
    return jnp.maximum(v, 0.0) + jnp.log1p(jnp.exp(-jnp.abs(v))) - _LOG2


def _node_body(x_ref, wi_ref, bi_ref, wj_ref, bj_ref, xi_ref, xj0_ref,
               xj1_ref):
    xa = _ssp(x_ref[...])
    xi = _ssp(jnp.dot(xa, wi_ref[...], preferred_element_type=jnp.float32)
              + bi_ref[...])
    xj = _ssp(jnp.dot(xa, wj_ref[...], preferred_element_type=jnp.float32)
              + bj_ref[...])
    xi_ref[...] = xi
    xj0_ref[...] = xj[:, :32]
    xj1_ref[...] = xj[:, 32:]


def _node_transform(x, wi, bi, wj, bj, bn):
    n, d = x.shape
    grid = (n // bn,)
    return pl.pallas_call(
        _node_body,
        grid=grid,
        in_specs=[
            pl.BlockSpec((bn, d), lambda i: (i, 0)),
            pl.BlockSpec((d, d), lambda i: (0, 0)),
            pl.BlockSpec((d,), lambda i: (0,)),
            pl.BlockSpec((d, d), lambda i: (0, 0)),
            pl.BlockSpec((d,), lambda i: (0,)),
        ],
        out_specs=[
            pl.BlockSpec((bn, d), lambda i: (i, 0)),
            pl.BlockSpec((bn, 32), lambda i: (i, 0)),
            pl.BlockSpec((bn, 32), lambda i: (i, 0)),
        ],
        out_shape=[
            jax.ShapeDtypeStruct((n, d), jnp.float32),
            jax.ShapeDtypeStruct((n, 32), jnp.float32),
            jax.ShapeDtypeStruct((n, 32), jnp.float32),
        ],
    )(x, wi, bi, wj, bj)


def _g_body(f0_ref, f1_ref, f2_ref, f3_ref, wa_ref, g0_ref, g1_ref):
    outs0, outs1 = [], []
    for fr in (f0_ref, f1_ref, f2_ref, f3_ref):
        g = jnp.dot(fr[...], wa_ref[...],
                    preferred_element_type=jnp.float32)
        outs0.append(g[:, :32])
        outs1.append(g[:, 32:])
    g0_ref[...] = jnp.concatenate(outs0, axis=1)
    g1_ref[...] = jnp.concatenate(outs1, axis=1)


def _g_transform(f_ij, w_att, bq):
    e, r = f_ij.shape
    d = w_att.shape[1]
    nblk = e // 4 // bq
    grid = (nblk,)

    def fmap(a):
        return lambda i: (i + a * nblk, 0)

    return pl.pallas_call(
        _g_body,
        grid=grid,
        in_specs=[
            pl.BlockSpec((bq, r), fmap(0)),
            pl.BlockSpec((bq, r), fmap(1)),
            pl.BlockSpec((bq, r), fmap(2)),
            pl.BlockSpec((bq, r), fmap(3)),
            pl.BlockSpec((r, d), lambda i: (0, 0)),
        ],
        out_specs=[
            pl.BlockSpec((bq, 128), lambda i: (i, 0)),
            pl.BlockSpec((bq, 128), lambda i: (i, 0)),
        ],
        out_shape=[
            jax.ShapeDtypeStruct((e // 4, 128), jnp.float32),
            jax.ShapeDtypeStruct((e // 4, 128), jnp.float32),
        ],
    )(f_ij, f_ij, f_ij, f_ij, w_att)


def _make_sc_edge(n, e):
    ept = e // _NS
    nwin = ept // _W
    nchunk = nwin // _WPC
    assert nwin == nchunk * _WPC and nwin >= _NBUF
    rpt = -((-n // _NS) // 8) * 8
    rlast = n - rpt * (_NS - 1)
    assert rlast > 0

    mesh = plsc.VectorSubcoreMesh(
        core_axis_name="c", subcore_axis_name="s",
        num_cores=_NC, num_subcores=_NS)

    @functools.partial(
        pl.kernel,
        mesh=mesh,
        compiler_params=pltpu.CompilerParams(use_tc_tiling_on_sc=False),
        out_type=jax.ShapeDtypeStruct((n, 128), jnp.float32),
        scratch_types=[
            pltpu.VMEM((_CH,), jnp.int32),
            pltpu.VMEM((_CH,), jnp.int32),
            pltpu.VMEM((_NBUF, _W), jnp.int32),
            pltpu.VMEM((_NBUF, _W, 32), jnp.float32),
            pltpu.VMEM((_NBUF, _W, 32), jnp.float32),
            pltpu.VMEM_SHARED((n, 32), jnp.float32),
            pltpu.SemaphoreType.DMA((_NBUF,)),
            pltpu.SemaphoreType.DMA((_NBUF,)),
        ],
    )
    def sc_edge(xj0_hbm, xj1_hbm, g0_hbm, g1_hbm,
                pair_hbm, zeros_hbm, out_hbm,
                ci_v, cj_v, idxi_v, rows_v, g_v, acc, sem_in, sem_sc):
        c = lax.axis_index("c")
        s = lax.axis_index("s")

        r0 = s * rpt

        @pl.when(s < _NS - 1)
        def _():
            pltpu.sync_copy(zeros_hbm.at[pl.ds(r0, rpt)],
                            acc.at[pl.ds(r0, rpt)])

        @pl.when(s == _NS - 1)
        def _():
            pltpu.sync_copy(zeros_hbm.at[pl.ds(r0, rlast)],
                            acc.at[pl.ds(r0, rlast)])

        plsc.subcore_barrier()

        ebase = s * ept

        def _drain(buf, slot, sem):
            pltpu.make_async_copy(zeros_hbm.at[pl.ds(0, _W)],
                                  buf.at[slot], sem).wait()

        def _run(table_hbm, gq_hbm, cidx):
            qa = s // 4
            gbase = ebase - qa * (e // 4)
            gcol = qa * 32

            def chunk_body(q, carry):
                cb = ebase + q * _CH
                pltpu.sync_copy(pair_hbm.at[0, pl.ds(cb, _CH)], ci_v)
                pltpu.sync_copy(pair_hbm.at[1, pl.ds(cb, _CH)], cj_v)

                def win_body(t, carry2):
                    gw = q * _WPC + t

                    @pl.when(t < _WPC)
                    def _():
                        slot = gw % _NBUF

                        @pl.when(gw >= _NBUF)
                        def _():
                            _drain(rows_v, slot, sem_sc.at[slot])

                        for i in range(_W // _L):
                            sl = pl.ds(i * _L, _L)
                            src = pl.ds(t * _W + i * _L, _L)
                            idxi_v[slot, sl] = ci_v[src]
                        pltpu.async_copy(
                            table_hbm.at[cj_v.at[pl.ds(t * _W, _W)]],
                            rows_v.at[slot], sem_in.at[slot])
                        pltpu.async_copy(
                            gq_hbm.at[pl.ds(gbase + q * _CH + t * _W, _W),
                                      pl.ds(gcol, 32)],
                            g_v.at[slot], sem_in.at[slot])

                    @pl.when(t >= 2)
                    def _():
                        slot = (gw - 2) % _NBUF
                        _drain(rows_v, slot, sem_in.at[slot])
                        _drain(g_v, slot, sem_in.at[slot])
                        for j in range(_W):
                            for h in range(2):
                                sl = pl.ds(h * _L, _L)
                                rows_v[slot, j, sl] = (rows_v[slot, j, sl]
                                                       * g_v[slot, j, sl])
                        pltpu.async_copy(rows_v.at[slot],
                                         acc.at[idxi_v.at[slot]],
                                         sem_sc.at[slot], add=True)

                    return carry2

                lax.fori_loop(0, _WPC + 2, win_body, 0)
                return carry

            lax.fori_loop(0, nchunk, chunk_body, 0)
            for b in range(_NBUF):
                _drain(rows_v, b, sem_sc.at[b])
            plsc.subcore_barrier()

            @pl.when(s < _NS - 1)
            def _():
                pltpu.sync_copy(acc.at[pl.ds(r0, rpt)],
                                out_hbm.at[pl.ds(r0, rpt),
                                           pl.ds(cidx * 32, 32)])

            @pl.when(s == _NS - 1)
            def _():
                pltpu.sync_copy(acc.at[pl.ds(r0, rlast)],
                                out_hbm.at[pl.ds(r0, rlast),
                                           pl.ds(cidx * 32, 32)])

        @pl.when(c == 0)
        def _():
            _run(xj0_hbm, g0_hbm, 0)

        @pl.when(c == 1)
        def _():
            _run(xj1_hbm, g1_hbm, 1)

    return sc_edge


def _post_body(x_ref, xi_ref, xjp_ref, dw_ref, db_ref, rw_ref, rb_ref,
               wv_ref, bv_ref, gate_ref, o_ref):
    m = xi_ref[...] + xjp_ref[...][:, :64]
    for l in range(3):
        t = _ssp(jnp.dot(m, dw_ref[l], preferred_element_type=jnp.float32)
                 + db_ref[l])
        m = m + jnp.dot(t, rw_ref[l], preferred_element_type=jnp.float32) \
              + rb_ref[l]
    o_ref[...] = (gate_ref[...] * x_ref[...]
                  + jnp.dot(_ssp(m), wv_ref[...],
                            preferred_element_type=jnp.float32)
                  + bv_ref[...])


def _post_transform(x, xi, xjp, dw, db, rw, rb, wv, bv, gate, bn):
    n, d = x.shape
    nres = dw.shape[0]
    grid = (n // bn,)
    return pl.pallas_call(
        _post_body,
        grid=grid,
        in_specs=[
            pl.BlockSpec((bn, d), lambda i: (i, 0)),
            pl.BlockSpec((bn, d), lambda i: (i, 0)),
            pl.BlockSpec((bn, 128), lambda i: (i, 0)),
            pl.BlockSpec((nres, d, d), lambda i: (0, 0, 0)),
            pl.BlockSpec((nres, d), lambda i: (0, 0)),
            pl.BlockSpec((nres, d, d), lambda i: (0, 0, 0)),
            pl.BlockSpec((nres, d), lambda i: (0, 0)),
            pl.BlockSpec((d, d), lambda i: (0, 0)),
            pl.BlockSpec((d,), lambda i: (0,)),
            pl.BlockSpec((d,), lambda i: (0,)),
        ],
        out_specs=pl.BlockSpec((bn, d), lambda i: (i, 0)),
        out_shape=jax.ShapeDtypeStruct((n, d), jnp.float32),
    )(x, xi, xjp, dw, db, rw, rb, wv, bv, gate)


def kernel(atomic_embedding, pair_indices, f_ij, W_att, Wi, bi, Wj, bj,
           Wv, bv, dW, db, rW, rb, gate):
    x = atomic_embedding
    n, d = x.shape
    e = f_ij.shape[0]
    idx_i = pair_indices[0]
    idx_j = pair_indices[1]

    bn = min(2000, n)
    bq = min(2000, e // 4)
    xi, xj0, xj1 = _node_transform(x, Wi, bi, Wj, bj, bn=bn)
    g0, g1 = _g_transform(f_ij, W_att, bq=bq)

    zeros = jnp.zeros((n, 32), jnp.float32)

    sc_edge = _make_sc_edge(n, e)
    xjp = sc_edge(xj0, xj1, g0, g1, pair_indices, zeros)

    return _post_transform(x, xi, xjp, dW, db, rW, rb, Wv, bv, gate, bn=bn)


# speedup vs baseline: 1.2474x; 1.0045x over previous
import functools

import jax
import jax.numpy as jnp
from jax import lax
from jax.experimental import pallas as pl
from jax.experimental.pallas import tpu as pltpu
from jax.experimental.pallas import tpu_sc as plsc

_LOG2 = 0.6931471805599453

_NC = 2
_NS = 16
_L = 16

_W = 80
_WPC = 25
_CH = _W * _WPC
_NBUF = 4
_LAG = 3


def _ssp(v):
    return jnp.maximum(v, 0.0) + jnp.log1p(jnp.exp(-jnp.abs(v))) - _LOG2


def _node_body(x_ref, wi_ref, bi_ref, wj_ref, bj_ref, xi_ref, xj0_ref,
               xj1_ref):
    xa = _ssp(x_ref[...])
    xi = _ssp(jnp.dot(xa, wi_ref[...], preferred_element_type=jnp.float32)
              + bi_ref[...])
    xj = _ssp(jnp.dot(xa, wj_ref[...], preferred_element_type=jnp.float32)
              + bj_ref[...])
    xi_ref[...] = xi
    xj0_ref[...] = xj[:, :32]
    xj1_ref[...] = xj[:, 32:]


def _node_transform(x, wi, bi, wj, bj, bn):
    n, d = x.shape
    grid = (n // bn,)
    return pl.pallas_call(
        _node_body,
        grid=grid,
        in_specs=[
            pl.BlockSpec((bn, d), lambda i: (i, 0)),
            pl.BlockSpec((d, d), lambda i: (0, 0)),
            pl.BlockSpec((d,), lambda i: (0,)),
            pl.BlockSpec((d, d), lambda i: (0, 0)),
            pl.BlockSpec((d,), lambda i: (0,)),
        ],
        out_specs=[
            pl.BlockSpec((bn, d), lambda i: (i, 0)),
            pl.BlockSpec((bn, 32), lambda i: (i, 0)),
            pl.BlockSpec((bn, 32), lambda i: (i, 0)),
        ],
        out_shape=[
            jax.ShapeDtypeStruct((n, d), jnp.float32),
            jax.ShapeDtypeStruct((n, 32), jnp.float32),
            jax.ShapeDtypeStruct((n, 32), jnp.float32),
        ],
    )(x, wi, bi, wj, bj)


def _g_body(f0_ref, f1_ref, f2_ref, f3_ref, wa_ref, g0_ref, g1_ref):
    outs0, outs1 = [], []
    for fr in (f0_ref, f1_ref, f2_ref, f3_ref):
        g = jnp.dot(fr[...], wa_ref[...],
                    preferred_element_type=jnp.float32)
        outs0.append(g[:, :32])
        outs1.append(g[:, 32:])
    g0_ref[...] = jnp.concatenate(outs0, axis=1)
    g1_ref[...] = jnp.concatenate(outs1, axis=1)


def _g_transform(f_ij, w_att, bq):
    e, r = f_ij.shape
    d = w_att.shape[1]
    nblk = e // 4 // bq
    grid = (nblk,)

    def fmap(a):
        return lambda i: (i + a * nblk, 0)

    return pl.pallas_call(
        _g_body,
        grid=grid,
        in_specs=[
            pl.BlockSpec((bq, r), fmap(0)),
            pl.BlockSpec((bq, r), fmap(1)),
            pl.BlockSpec((bq, r), fmap(2)),
            pl.BlockSpec((bq, r), fmap(3)),
            pl.BlockSpec((r, d), lambda i: (0, 0)),
        ],
        out_specs=[
            pl.BlockSpec((bq, 128), lambda i: (i, 0)),
            pl.BlockSpec((bq, 128), lambda i: (i, 0)),
        ],
        out_shape=[
            jax.ShapeDtypeStruct((e // 4, 128), jnp.float32),
            jax.ShapeDtypeStruct((e // 4, 128), jnp.float32),
        ],
    )(f_ij, f_ij, f_ij, f_ij, w_att)


def _make_sc_edge(n, e):
    ept = e // _NS
    nwin = ept // _W
    nchunk = nwin // _WPC
    assert nwin == nchunk * _WPC and nwin >= _NBUF
    rpt = -((-n // _NS) // 8) * 8
    rlast = n - rpt * (_NS - 1)
    assert rlast > 0

    mesh = plsc.VectorSubcoreMesh(
        core_axis_name="c", subcore_axis_name="s",
        num_cores=_NC, num_subcores=_NS)

    @functools.partial(
        pl.kernel,
        mesh=mesh,
        compiler_params=pltpu.CompilerParams(use_tc_tiling_on_sc=False),
        out_type=jax.ShapeDtypeStruct((n, 128), jnp.float32),
        scratch_types=[
            pltpu.VMEM((_CH,), jnp.int32),
            pltpu.VMEM((_CH,), jnp.int32),
            pltpu.VMEM((_NBUF, _W), jnp.int32),
            pltpu.VMEM((_NBUF, _W, 32), jnp.float32),
            pltpu.VMEM((_NBUF, _W, 32), jnp.float32),
            pltpu.VMEM_SHARED((n, 32), jnp.float32),
            pltpu.SemaphoreType.DMA((_NBUF,)),
            pltpu.SemaphoreType.DMA((_NBUF,)),
        ],
    )
    def sc_edge(xj0_hbm, xj1_hbm, g0_hbm, g1_hbm,
                pair_hbm, zeros_hbm, out_hbm,
                ci_v, cj_v, idxi_v, rows_v, g_v, acc, sem_in, sem_sc):
        c = lax.axis_index("c")
        s = lax.axis_index("s")

        r0 = s * rpt

        @pl.when(s < _NS - 1)
        def _():
            pltpu.sync_copy(zeros_hbm.at[pl.ds(r0, rpt)],
                            acc.at[pl.ds(r0, rpt)])

        @pl.when(s == _NS - 1)
        def _():
            pltpu.sync_copy(zeros_hbm.at[pl.ds(r0, rlast)],
                            acc.at[pl.ds(r0, rlast)])

        plsc.subcore_barrier()

        ebase = s * ept

        def _drain(buf, slot, sem):
            pltpu.make_async_copy(zeros_hbm.at[pl.ds(0, _W)],
                                  buf.at[slot], sem).wait()

        def _run(table_hbm, gq_hbm, cidx):
            qa = s // 4
            gbase = ebase - qa * (e // 4)
            gcol = qa * 32

            def chunk_body(q, carry):
                cb = ebase + q * _CH
                pltpu.sync_copy(pair_hbm.at[0, pl.ds(cb, _CH)], ci_v)
                pltpu.sync_copy(pair_hbm.at[1, pl.ds(cb, _CH)], cj_v)

                def win_body(t, carry2):
                    gw = q * _WPC + t

                    @pl.when(t < _WPC)
                    def _():
                        slot = gw % _NBUF

                        @pl.when(gw >= _NBUF)
                        def _():
                            _drain(rows_v, slot, sem_sc.at[slot])

                        for i in range(_W // _L):
                            sl = pl.ds(i * _L, _L)
                            src = pl.ds(t * _W + i * _L, _L)
                            idxi_v[slot, sl] = ci_v[src]
                        pltpu.async_copy(
                            table_hbm.at[cj_v.at[pl.ds(t * _W, _W)]],
                            rows_v.at[slot], sem_in.at[slot])
                        pltpu.async_copy(
                            gq_hbm.at[pl.ds(gbase + q * _CH + t * _W, _W),
                                      pl.ds(gcol, 32)],
                            g_v.at[slot], sem_in.at[slot])

                    @pl.when(t >= _LAG)
                    def _():
                        slot = (gw - _LAG) % _NBUF
                        _drain(rows_v, slot, sem_in.at[slot])
                        _drain(g_v, slot, sem_in.at[slot])
                        for j in range(_W):
                            for h in range(2):
                                sl = pl.ds(h * _L, _L)
                                rows_v[slot, j, sl] = (rows_v[slot, j, sl]
                                                       * g_v[slot, j, sl])
                        pltpu.async_copy(rows_v.at[slot],
                                         acc.at[idxi_v.at[slot]],
                                         sem_sc.at[slot], add=True)

                    return carry2

                lax.fori_loop(0, _WPC + _LAG, win_body, 0)
                return carry

            lax.fori_loop(0, nchunk, chunk_body, 0)
            for b in range(_NBUF):
                _drain(rows_v, b, sem_sc.at[b])
            plsc.subcore_barrier()

            @pl.when(s < _NS - 1)
            def _():
                pltpu.sync_copy(acc.at[pl.ds(r0, rpt)],
                                out_hbm.at[pl.ds(r0, rpt),
                                           pl.ds(cidx * 32, 32)])

            @pl.when(s == _NS - 1)
            def _():
                pltpu.sync_copy(acc.at[pl.ds(r0, rlast)],
                                out_hbm.at[pl.ds(r0, rlast),
                                           pl.ds(cidx * 32, 32)])

        @pl.when(c == 0)
        def _():
            _run(xj0_hbm, g0_hbm, 0)

        @pl.when(c == 1)
        def _():
            _run(xj1_hbm, g1_hbm, 1)

    return sc_edge


def _post_body(x_ref, xi_ref, xjp_ref, dw_ref, db_ref, rw_ref, rb_ref,
               wv_ref, bv_ref, gate_ref, o_ref):
    m = xi_ref[...] + xjp_ref[...][:, :64]
    for l in range(3):
        t = _ssp(jnp.dot(m, dw_ref[l], preferred_element_type=jnp.float32)
                 + db_ref[l])
        m = m + jnp.dot(t, rw_ref[l], preferred_element_type=jnp.float32) \
              + rb_ref[l]
    o_ref[...] = (gate_ref[...] * x_ref[...]
                  + jnp.dot(_ssp(m), wv_ref[...],
                            preferred_element_type=jnp.float32)
                  + bv_ref[...])


def _post_transform(x, xi, xjp, dw, db, rw, rb, wv, bv, gate, bn):
    n, d = x.shape
    nres = dw.shape[0]
    grid = (n // bn,)
    return pl.pallas_call(
        _post_body,
        grid=grid,
        in_specs=[
            pl.BlockSpec((bn, d), lambda i: (i, 0)),
            pl.BlockSpec((bn, d), lambda i: (i, 0)),
            pl.BlockSpec((bn, 128), lambda i: (i, 0)),
            pl.BlockSpec((nres, d, d), lambda i: (0, 0, 0)),
            pl.BlockSpec((nres, d), lambda i: (0, 0)),
            pl.BlockSpec((nres, d, d), lambda i: (0, 0, 0)),
            pl.BlockSpec((nres, d), lambda i: (0, 0)),
            pl.BlockSpec((d, d), lambda i: (0, 0)),
            pl.BlockSpec((d,), lambda i: (0,)),
            pl.BlockSpec((d,), lambda i: (0,)),
        ],
        out_specs=pl.BlockSpec((bn, d), lambda i: (i, 0)),
        out_shape=jax.ShapeDtypeStruct((n, d), jnp.float32),
    )(x, xi, xjp, dw, db, rw, rb, wv, bv, gate)


def kernel(atomic_embedding, pair_indices, f_ij, W_att, Wi, bi, Wj, bj,
           Wv, bv, dW, db, rW, rb, gate):
    x = atomic_embedding
    n, d = x.shape
    e = f_ij.shape[0]
    idx_i = pair_indices[0]
    idx_j = pair_indices[1]

    bn = min(2000, n)
    bq = min(2000, e // 4)
    xi, xj0, xj1 = _node_transform(x, Wi, bi, Wj, bj, bn=bn)
    g0, g1 = _g_transform(f_ij, W_att, bq=bq)

    zeros = jnp.zeros((n, 32), jnp.float32)

    sc_edge = _make_sc_edge(n, e)
    xjp = sc_edge(xj0, xj1, g0, g1, pair_indices, zeros)

    return _post_transform(x, xi, xjp, dW, db, rW, rb, Wv, bv, gate, bn=bn)
